# baseline (speedup 1.0000x reference)
import functools

import jax
import jax.numpy as jnp
from jax import lax
from jax.experimental import pallas as pl
from jax.experimental.pallas import tpu as pltpu
from jax.experimental.pallas import tpu_sc as plsc

N_HEADS = 2
N = 10000
E = 320000
F = 128
FH = 64
SPW = 16
RW = FH + SPW

NC, NS = 2, 16
EPS = E // NS
CH = 80
NCHUNK = EPS // CH


def _prep_body(x_ref, w_ref, amat_ref, haug_ref, sp_ref):
    b = x_ref.shape[0]
    h = x_ref[...] * w_ref[...]
    s = jnp.dot(h, amat_ref[...], preferred_element_type=jnp.float32)
    col = lax.broadcasted_iota(jnp.int32, (b, SPW), 1)
    sp = s + (col == 6).astype(jnp.float32)
    haug_ref[0] = jnp.concatenate([h[:, :FH], sp], axis=1)
    haug_ref[1] = jnp.concatenate([h[:, FH:], sp], axis=1)
    sp_ref[...] = sp


def _fin_body(h0l, h0r, h1l, h1r, out_ref):
    l0 = h0l[...]
    r0 = h0r[...]
    out_ref[0] = (jnp.concatenate([l0[:, :FH], r0[:, :FH]], axis=1)
                  / l0[:, FH:FH + 1])
    out_ref[1] = (jnp.concatenate([h1l[...], h1r[...]], axis=1)
                  / l0[:, FH + 1:FH + 2])


def _edge_kernel(haug_hbm, sp_hbm, a0_hbm, a2_hbm, r1_hbm,
                 rawf0_hbm, rawf1_hbm,
                 a0_v, a2_v, r1_v, rows_v, spa0_v, spr_v,
                 valf0_v, valf1_v,
                 accf0, accf1,
                 sem0a, sem1a, sem2a, sem0b, sem1b, sem2b,
                 isema, isemb):
    c_id = lax.axis_index("c")
    s_id = lax.axis_index("s")
    zero16 = jnp.zeros((16,), jnp.float32)
    lane = lax.iota(jnp.int32, 16)

    def zrow(r, _):
        for j in range(RW // 16):
            valf0_v[r, pl.ds(j * 16, 16)] = zero16
        for j in range(FH // 16):
            valf1_v[r, pl.ds(j * 16, 16)] = zero16
        return 0
    lax.fori_loop(0, CH, zrow, 0)

    @pl.when(s_id < 10)
    def _():
        u0 = s_id * 1000
        for k in range(1000 // 40):
            dst = pl.ds(u0 + k * 40, 40)
            pltpu.sync_copy(valf0_v.at[pl.ds(0, 40)], accf0.at[dst])

    @pl.when(s_id >= 6)
    def _():
        u0 = (s_id - 6) * 1000
        for k in range(1000 // 40):
            dst = pl.ds(u0 + k * 40, 40)
            pltpu.sync_copy(valf1_v.at[pl.ds(0, 40)], accf1.at[dst])

    plsc.subcore_barrier()

    cbase = c_id * N
    sems = ((sem0a, sem1a, sem2a), (sem0b, sem1b, sem2b))
    isems = (isema, isemb)

    def idx_slices(g):
        base = s_id * EPS + g * CH
        return pl.ds(base, CH)

    def issue_idx(g, p):
        esl = idx_slices(g)
        pltpu.async_copy(a0_hbm.at[esl], a0_v.at[p], isems[p])
        pltpu.async_copy(a2_hbm.at[esl], a2_v.at[p], isems[p])
        pltpu.async_copy(r1_hbm.at[esl], r1_v.at[p], isems[p])

    def wait_idx(g, p):
        esl = idx_slices(g)
        pltpu.make_async_copy(a0_hbm.at[esl], a0_v.at[p], isems[p]).wait()
        pltpu.make_async_copy(a2_hbm.at[esl], a2_v.at[p], isems[p]).wait()
        pltpu.make_async_copy(r1_hbm.at[esl], r1_v.at[p], isems[p]).wait()

    def issue_gathers(p):
        for j in range(CH // 16):
            sl = pl.ds(j * 16, 16)
            a2_v[p, sl] = a2_v[p, sl] + cbase
        pltpu.async_copy(haug_hbm.at[a2_v.at[p]], rows_v.at[p], sems[p][0])
        pltpu.async_copy(sp_hbm.at[a0_v.at[p]], spa0_v.at[p], sems[p][1])
        pltpu.async_copy(sp_hbm.at[r1_v.at[p]], spr_v.at[p], sems[p][2])

    def wait_gathers(p):
        pltpu.make_async_copy(
            haug_hbm.at[a2_v.at[p]], rows_v.at[p], sems[p][0]).wait()
        pltpu.make_async_copy(
            sp_hbm.at[a0_v.at[p]], spa0_v.at[p], sems[p][1]).wait()
        pltpu.make_async_copy(
            sp_hbm.at[r1_v.at[p]], spr_v.at[p], sems[p][2]).wait()

    def compute_scatter(p):
        UNROLL = 4

        def edge(u, _):
            c0 = u * UNROLL
            ebs = []
            for q in range(UNROLL):
                c = c0 + q
                spa0row = spa0_v[p, c, pl.ds(0, 16)]
                sprrow = spr_v[p, c, pl.ds(0, 16)]
                spx = rows_v[p, c, pl.ds(FH, 16)]
                eh0 = spa0row[0] + spx[1] + sprrow[2]
                eh1 = spa0row[3] + spx[4] + sprrow[5]
                eh0v = jnp.broadcast_to(eh0, (16,))
                eh1v = jnp.broadcast_to(eh1, (16,))
                e0b = jnp.exp(-jnp.where(eh0v >= 0, eh0v, 0.2 * eh0v))
                e1b = jnp.exp(-jnp.where(eh1v >= 0, eh1v, 0.2 * eh1v))
                ebs.append((c, e0b, e1b))
            for c, e0b, e1b in ebs:
                for j in range(FH // 16):
                    sl = pl.ds(j * 16, 16)
                    row = rows_v[p, c, sl]
                    valf0_v[c, sl] = row * e0b
                    valf1_v[c, sl] = row * e1b
                ve = jnp.where(lane == 0, e0b,
                               jnp.where(lane == 1, e1b, 0.0))
                valf0_v[c, pl.ds(FH, 16)] = ve
            return 0
        lax.fori_loop(0, CH // UNROLL, edge, 0)

        idx = a0_v.at[p]
        pltpu.sync_copy(valf0_v, accf0.at[idx], add=True)
        pltpu.sync_copy(valf1_v, accf1.at[idx], add=True)

    wait_idx_done = wait_idx

    def round_body(g, p):
        @pl.when(g + 1 < NCHUNK)
        def _():
            wait_idx_done(g + 1, 1 - p)
            issue_gathers(1 - p)
        wait_gathers(p)
        compute_scatter(p)

        @pl.when(g + 2 < NCHUNK)
        def _():
            issue_idx(g + 2, p)

    issue_idx(0, 0)
    wait_idx(0, 0)
    issue_gathers(0)
    issue_idx(1, 1)

    def pair(gp, _):
        g0 = gp * 2
        round_body(g0, 0)
        round_body(g0 + 1, 1)
        return 0

    lax.fori_loop(0, NCHUNK // 2, pair, 0)
    plsc.subcore_barrier()

    @pl.when(s_id < 10)
    def _():
        src = s_id * 1000
        off = c_id * N + src
        pltpu.sync_copy(accf0.at[pl.ds(src, 1000)],
                        rawf0_hbm.at[pl.ds(off, 1000)])

    @pl.when(s_id >= 6)
    def _():
        src = (s_id - 6) * 1000
        off = c_id * N + src
        pltpu.sync_copy(accf1.at[pl.ds(src, 1000)],
                        rawf1_hbm.at[pl.ds(off, 1000)])


def kernel(input, inputr, A, w, a_src_dst):
    x = input.astype(jnp.float32)
    a0 = A[0].astype(jnp.int32)
    a2 = A[2].astype(jnp.int32)
    r1 = inputr[1].astype(jnp.int32)
    w0 = w[0].astype(jnp.float32).reshape(1, F)
    amat = jnp.swapaxes(
        a_src_dst.astype(jnp.float32)[:, :, :, 0].reshape(6, F), 0, 1)
    amat = jnp.pad(amat, ((0, 0), (0, SPW - 6)))

    b0 = 1000
    haug, sp = pl.pallas_call(
        _prep_body,
        grid=(N // b0,),
        in_specs=[
            pl.BlockSpec((b0, F), lambda i: (i, 0)),
            pl.BlockSpec((1, F), lambda i: (0, 0)),
            pl.BlockSpec((F, SPW), lambda i: (0, 0)),
        ],
        out_specs=[
            pl.BlockSpec((2, b0, RW), lambda i: (0, i, 0)),
            pl.BlockSpec((b0, SPW), lambda i: (i, 0)),
        ],
        out_shape=[
            jax.ShapeDtypeStruct((2, N, RW), jnp.float32),
            jax.ShapeDtypeStruct((N, SPW), jnp.float32),
        ],
    )(x, w0, amat)
    haug_flat = haug.reshape(2 * N, RW)

    mesh = plsc.VectorSubcoreMesh(
        core_axis_name="c", subcore_axis_name="s",
        num_cores=NC, num_subcores=NS)
    edge_call = functools.partial(
        pl.kernel,
        out_type=[
            jax.ShapeDtypeStruct((NC * N, RW), jnp.float32),
            jax.ShapeDtypeStruct((NC * N, FH), jnp.float32),
        ],
        mesh=mesh,
        scratch_types=[
            pltpu.VMEM((2, CH), jnp.int32),
            pltpu.VMEM((2, CH), jnp.int32),
            pltpu.VMEM((2, CH), jnp.int32),
            pltpu.VMEM((2, CH, RW), jnp.float32),
            pltpu.VMEM((2, CH, SPW), jnp.float32),
            pltpu.VMEM((2, CH, SPW), jnp.float32),
            pltpu.VMEM((CH, RW), jnp.float32),
            pltpu.VMEM((CH, FH), jnp.float32),
            pltpu.VMEM_SHARED((N, RW), jnp.float32),
            pltpu.VMEM_SHARED((N, FH), jnp.float32),
            pltpu.SemaphoreType.DMA,
            pltpu.SemaphoreType.DMA,
            pltpu.SemaphoreType.DMA,
            pltpu.SemaphoreType.DMA,
            pltpu.SemaphoreType.DMA,
            pltpu.SemaphoreType.DMA,
            pltpu.SemaphoreType.DMA,
            pltpu.SemaphoreType.DMA,
        ],
        compiler_params=pltpu.CompilerParams(use_tc_tiling_on_sc=False),
    )(_edge_kernel)
    rawf0, rawf1 = edge_call(haug_flat, sp, a0, a2, r1)

    b2 = 1000
    nb = N // b2
    out = pl.pallas_call(
        _fin_body,
        grid=(nb,),
        in_specs=[
            pl.BlockSpec((b2, RW), lambda i: (i, 0)),
            pl.BlockSpec((b2, RW), lambda i: (nb + i, 0)),
            pl.BlockSpec((b2, FH), lambda i: (i, 0)),
            pl.BlockSpec((b2, FH), lambda i: (nb + i, 0)),
        ],
        out_specs=pl.BlockSpec((2, b2, F), lambda i: (0, i, 0)),
        out_shape=jax.ShapeDtypeStruct((N_HEADS, N, F), jnp.float32),
    )(rawf0, rawf0, rawf1, rawf1)
    return out


# speedup vs baseline: 1.0622x; 1.0622x over previous
import functools

import jax
import jax.numpy as jnp
from jax import lax
from jax.experimental import pallas as pl
from jax.experimental.pallas import tpu as pltpu
from jax.experimental.pallas import tpu_sc as plsc

N_HEADS = 2
N = 10000
E = 320000
F = 128
FH = 64
SPW = 16
RW = FH + SPW

NC, NS = 2, 16
EPS = E // NS
CH = 80
NCHUNK = EPS // CH


def _prep_body(x_ref, w_ref, amat_ref, haug_ref, sp_ref):
    b = x_ref.shape[0]
    h = x_ref[...] * w_ref[...]
    s = jnp.dot(h, amat_ref[...], preferred_element_type=jnp.float32)
    col = lax.broadcasted_iota(jnp.int32, (b, SPW), 1)
    sp = s + (col == 6).astype(jnp.float32)
    haug_ref[0] = jnp.concatenate([h[:, :FH], sp], axis=1)
    haug_ref[1] = jnp.concatenate([h[:, FH:], sp], axis=1)
    sp_ref[...] = sp


def _fin_body(h0l, h0r, h1l, h1r, out_ref):
    l0 = h0l[...]
    r0 = h0r[...]
    out_ref[0] = (jnp.concatenate([l0[:, :FH], r0[:, :FH]], axis=1)
                  / l0[:, FH:FH + 1])
    out_ref[1] = (jnp.concatenate([h1l[...], h1r[...]], axis=1)
                  / l0[:, FH + 1:FH + 2])


def _edge_kernel(haug_hbm, sp_hbm, a0_hbm, a2_hbm, r1_hbm,
                 rawf0_hbm, rawf1_hbm,
                 a0_v, a2_v, r1_v, rows_v, spa0_v, spr_v,
                 valf0_v, valf1_v,
                 accf0, accf1,
                 sem0a, sem1a, sem2a, sem0b, sem1b, sem2b,
                 isema, isemb):
    c_id = lax.axis_index("c")
    s_id = lax.axis_index("s")
    zero16 = jnp.zeros((16,), jnp.float32)
    lane = lax.iota(jnp.int32, 16)

    def zrow(r, _):
        for j in range(RW // 16):
            valf0_v[r, pl.ds(j * 16, 16)] = zero16
        for j in range(FH // 16):
            valf1_v[r, pl.ds(j * 16, 16)] = zero16
        return 0
    lax.fori_loop(0, CH, zrow, 0)

    @pl.when(s_id < 10)
    def _():
        u0 = s_id * 1000
        for k in range(1000 // 40):
            dst = pl.ds(u0 + k * 40, 40)
            pltpu.sync_copy(valf0_v.at[pl.ds(0, 40)], accf0.at[dst])

    @pl.when(s_id >= 6)
    def _():
        u0 = (s_id - 6) * 1000
        for k in range(1000 // 40):
            dst = pl.ds(u0 + k * 40, 40)
            pltpu.sync_copy(valf1_v.at[pl.ds(0, 40)], accf1.at[dst])

    plsc.subcore_barrier()

    cbase = c_id * N
    sems = ((sem0a, sem1a, sem2a), (sem0b, sem1b, sem2b))
    isems = (isema, isemb)

    def idx_slices(g):
        base = s_id * EPS + g * CH
        return pl.ds(base, CH)

    def issue_idx(g, p):
        esl = idx_slices(g)
        pltpu.async_copy(a0_hbm.at[esl], a0_v.at[p], isems[p])
        pltpu.async_copy(a2_hbm.at[esl], a2_v.at[p], isems[p])
        pltpu.async_copy(r1_hbm.at[esl], r1_v.at[p], isems[p])

    def wait_idx(g, p):
        esl = idx_slices(g)
        pltpu.make_async_copy(a0_hbm.at[esl], a0_v.at[p], isems[p]).wait()
        pltpu.make_async_copy(a2_hbm.at[esl], a2_v.at[p], isems[p]).wait()
        pltpu.make_async_copy(r1_hbm.at[esl], r1_v.at[p], isems[p]).wait()

    def issue_gathers(p):
        for j in range(CH // 16):
            sl = pl.ds(j * 16, 16)
            a2_v[p, sl] = a2_v[p, sl] + cbase
        pltpu.async_copy(haug_hbm.at[a2_v.at[p]], rows_v.at[p], sems[p][0])
        pltpu.async_copy(sp_hbm.at[a0_v.at[p]], spa0_v.at[p], sems[p][1])
        pltpu.async_copy(sp_hbm.at[r1_v.at[p]], spr_v.at[p], sems[p][2])

    def wait_gathers(p):
        pltpu.make_async_copy(
            haug_hbm.at[a2_v.at[p]], rows_v.at[p], sems[p][0]).wait()
        pltpu.make_async_copy(
            sp_hbm.at[a0_v.at[p]], spa0_v.at[p], sems[p][1]).wait()
        pltpu.make_async_copy(
            sp_hbm.at[r1_v.at[p]], spr_v.at[p], sems[p][2]).wait()

    def compute_scatter(p):
        UNROLL = 8

        def edge(u, _):
            c0 = u * UNROLL
            ebs = []
            for q in range(UNROLL):
                c = c0 + q
                spa0row = spa0_v[p, c, pl.ds(0, 16)]
                sprrow = spr_v[p, c, pl.ds(0, 16)]
                spx = rows_v[p, c, pl.ds(FH, 16)]
                eh0 = spa0row[0] + spx[1] + sprrow[2]
                eh1 = spa0row[3] + spx[4] + sprrow[5]
                eh0v = jnp.broadcast_to(eh0, (16,))
                eh1v = jnp.broadcast_to(eh1, (16,))
                e0b = jnp.exp(-jnp.where(eh0v >= 0, eh0v, 0.2 * eh0v))
                e1b = jnp.exp(-jnp.where(eh1v >= 0, eh1v, 0.2 * eh1v))
                ebs.append((c, e0b, e1b))
            for c, e0b, e1b in ebs:
                for j in range(FH // 16):
                    sl = pl.ds(j * 16, 16)
                    row = rows_v[p, c, sl]
                    valf0_v[c, sl] = row * e0b
                    valf1_v[c, sl] = row * e1b
                ve = jnp.where(lane == 0, e0b,
                               jnp.where(lane == 1, e1b, 0.0))
                valf0_v[c, pl.ds(FH, 16)] = ve
            return 0
        lax.fori_loop(0, CH // UNROLL, edge, 0)

        idx = a0_v.at[p]
        pltpu.sync_copy(valf0_v, accf0.at[idx], add=True)
        pltpu.sync_copy(valf1_v, accf1.at[idx], add=True)

    wait_idx_done = wait_idx

    def round_body(g, p):
        @pl.when(g + 1 < NCHUNK)
        def _():
            wait_idx_done(g + 1, 1 - p)
            issue_gathers(1 - p)
        wait_gathers(p)
        compute_scatter(p)

        @pl.when(g + 2 < NCHUNK)
        def _():
            issue_idx(g + 2, p)

    issue_idx(0, 0)
    wait_idx(0, 0)
    issue_gathers(0)
    issue_idx(1, 1)

    def pair(gp, _):
        g0 = gp * 2
        round_body(g0, 0)
        round_body(g0 + 1, 1)
        return 0

    lax.fori_loop(0, NCHUNK // 2, pair, 0)
    plsc.subcore_barrier()

    @pl.when(s_id < 10)
    def _():
        src = s_id * 1000
        off = c_id * N + src
        pltpu.sync_copy(accf0.at[pl.ds(src, 1000)],
                        rawf0_hbm.at[pl.ds(off, 1000)])

    @pl.when(s_id >= 6)
    def _():
        src = (s_id - 6) * 1000
        off = c_id * N + src
        pltpu.sync_copy(accf1.at[pl.ds(src, 1000)],
                        rawf1_hbm.at[pl.ds(off, 1000)])


def kernel(input, inputr, A, w, a_src_dst):
    x = input.astype(jnp.float32)
    a0 = A[0].astype(jnp.int32)
    a2 = A[2].astype(jnp.int32)
    r1 = inputr[1].astype(jnp.int32)
    w0 = w[0].astype(jnp.float32).reshape(1, F)
    amat = jnp.swapaxes(
        a_src_dst.astype(jnp.float32)[:, :, :, 0].reshape(6, F), 0, 1)
    amat = jnp.pad(amat, ((0, 0), (0, SPW - 6)))

    b0 = 1000
    haug, sp = pl.pallas_call(
        _prep_body,
        grid=(N // b0,),
        in_specs=[
            pl.BlockSpec((b0, F), lambda i: (i, 0)),
            pl.BlockSpec((1, F), lambda i: (0, 0)),
            pl.BlockSpec((F, SPW), lambda i: (0, 0)),
        ],
        out_specs=[
            pl.BlockSpec((2, b0, RW), lambda i: (0, i, 0)),
            pl.BlockSpec((b0, SPW), lambda i: (i, 0)),
        ],
        out_shape=[
            jax.ShapeDtypeStruct((2, N, RW), jnp.float32),
            jax.ShapeDtypeStruct((N, SPW), jnp.float32),
        ],
    )(x, w0, amat)
    haug_flat = haug.reshape(2 * N, RW)

    mesh = plsc.VectorSubcoreMesh(
        core_axis_name="c", subcore_axis_name="s",
        num_cores=NC, num_subcores=NS)
    edge_call = functools.partial(
        pl.kernel,
        out_type=[
            jax.ShapeDtypeStruct((NC * N, RW), jnp.float32),
            jax.ShapeDtypeStruct((NC * N, FH), jnp.float32),
        ],
        mesh=mesh,
        scratch_types=[
            pltpu.VMEM((2, CH), jnp.int32),
            pltpu.VMEM((2, CH), jnp.int32),
            pltpu.VMEM((2, CH), jnp.int32),
            pltpu.VMEM((2, CH, RW), jnp.float32),
            pltpu.VMEM((2, CH, SPW), jnp.float32),
            pltpu.VMEM((2, CH, SPW), jnp.float32),
            pltpu.VMEM((CH, RW), jnp.float32),
            pltpu.VMEM((CH, FH), jnp.float32),
            pltpu.VMEM_SHARED((N, RW), jnp.float32),
            pltpu.VMEM_SHARED((N, FH), jnp.float32),
            pltpu.SemaphoreType.DMA,
            pltpu.SemaphoreType.DMA,
            pltpu.SemaphoreType.DMA,
            pltpu.SemaphoreType.DMA,
            pltpu.SemaphoreType.DMA,
            pltpu.SemaphoreType.DMA,
            pltpu.SemaphoreType.DMA,
            pltpu.SemaphoreType.DMA,
        ],
        compiler_params=pltpu.CompilerParams(use_tc_tiling_on_sc=False),
    )(_edge_kernel)
    rawf0, rawf1 = edge_call(haug_flat, sp, a0, a2, r1)

    b2 = 1000
    nb = N // b2
    out = pl.pallas_call(
        _fin_body,
        grid=(nb,),
        in_specs=[
            pl.BlockSpec((b2, RW), lambda i: (i, 0)),
            pl.BlockSpec((b2, RW), lambda i: (nb + i, 0)),
            pl.BlockSpec((b2, FH), lambda i: (i, 0)),
            pl.BlockSpec((b2, FH), lambda i: (nb + i, 0)),
        ],
        out_specs=pl.BlockSpec((2, b2, F), lambda i: (0, i, 0)),
        out_shape=jax.ShapeDtypeStruct((N_HEADS, N, F), jnp.float32),
    )(rawf0, rawf0, rawf1, rawf1)
    return out


# speedup vs baseline: 1.6565x; 1.5596x over previous
import functools

import jax
import jax.numpy as jnp
from jax import lax
from jax.experimental import pallas as pl
from jax.experimental.pallas import tpu as pltpu
from jax.experimental.pallas import tpu_sc as plsc

N_HEADS = 2
N = 10000
E = 320000
F = 128
FH = 64
SPW = 16
RW = FH + SPW

NC, NS = 2, 16
EPS = E // NS
CH = 80
NCHUNK = EPS // CH


def _prep_body(x_ref, w_ref, am0_ref, am1_ref, am2_ref,
               haug_ref, spa_ref, spr_ref):
    h = x_ref[...] * w_ref[...]
    s0 = jnp.dot(h, am0_ref[...], preferred_element_type=jnp.float32)
    s1 = jnp.dot(h, am1_ref[...], preferred_element_type=jnp.float32)
    s2 = jnp.dot(h, am2_ref[...], preferred_element_type=jnp.float32)
    haug_ref[0] = jnp.concatenate([h[:, :FH], s1], axis=1)
    haug_ref[1] = jnp.concatenate([h[:, FH:], s1], axis=1)
    spa_ref[...] = s0
    spr_ref[...] = s2


def _fin_body(h0l, h0r, h1l, h1r, out_ref):
    l0 = h0l[...]
    r0 = h0r[...]
    out_ref[0] = (jnp.concatenate([l0[:, :FH], r0[:, :FH]], axis=1)
                  / l0[:, FH:FH + 1])
    out_ref[1] = (jnp.concatenate([h1l[...], h1r[...]], axis=1)
                  / l0[:, FH + 1:FH + 2])


def _edge_kernel(haug_hbm, spa_hbm, sprt_hbm, a0_hbm, a2_hbm, r1_hbm,
                 rawf0_hbm, rawf1_hbm,
                 a0_v, a2_v, r1_v, rows_v, spa0_v, spr_v,
                 valf0_v, valf1_v,
                 accf0, accf1,
                 sem0a, sem1a, sem2a, sem0b, sem1b, sem2b,
                 isema, isemb):
    c_id = lax.axis_index("c")
    s_id = lax.axis_index("s")
    zero16 = jnp.zeros((16,), jnp.float32)
    lane = lax.iota(jnp.int32, 16)

    def zrow(r, _):
        for j in range(RW // 16):
            valf0_v[r, pl.ds(j * 16, 16)] = zero16
        for j in range(FH // 16):
            valf1_v[r, pl.ds(j * 16, 16)] = zero16
        return 0
    lax.fori_loop(0, CH, zrow, 0)

    @pl.when(s_id < 10)
    def _():
        u0 = s_id * 1000
        for k in range(1000 // 40):
            dst = pl.ds(u0 + k * 40, 40)
            pltpu.sync_copy(valf0_v.at[pl.ds(0, 40)], accf0.at[dst])

    @pl.when(s_id >= 6)
    def _():
        u0 = (s_id - 6) * 1000
        for k in range(1000 // 40):
            dst = pl.ds(u0 + k * 40, 40)
            pltpu.sync_copy(valf1_v.at[pl.ds(0, 40)], accf1.at[dst])

    plsc.subcore_barrier()

    cbase = c_id * N
    sems = ((sem0a, sem1a, sem2a), (sem0b, sem1b, sem2b))
    isems = (isema, isemb)

    def idx_slices(g):
        base = s_id * EPS + g * CH
        return pl.ds(base, CH)

    def issue_idx(g, p):
        esl = idx_slices(g)
        pltpu.async_copy(a0_hbm.at[esl], a0_v.at[p], isems[p])
        pltpu.async_copy(a2_hbm.at[esl], a2_v.at[p], isems[p])
        pltpu.async_copy(r1_hbm.at[esl], r1_v.at[p], isems[p])

    def wait_idx(g, p):
        esl = idx_slices(g)
        pltpu.make_async_copy(a0_hbm.at[esl], a0_v.at[p], isems[p]).wait()
        pltpu.make_async_copy(a2_hbm.at[esl], a2_v.at[p], isems[p]).wait()
        pltpu.make_async_copy(r1_hbm.at[esl], r1_v.at[p], isems[p]).wait()

    def issue_gathers(p):
        for j in range(CH // 16):
            sl = pl.ds(j * 16, 16)
            a2_v[p, sl] = a2_v[p, sl] + cbase
        pltpu.async_copy(haug_hbm.at[a2_v.at[p]], rows_v.at[p], sems[p][0])
        pltpu.async_copy(spa_hbm.at[a0_v.at[p]], spa0_v.at[p], sems[p][1])
        pltpu.async_copy(sprt_hbm.at[r1_v.at[p]], spr_v.at[p], sems[p][2])

    def wait_gathers(p):
        pltpu.make_async_copy(
            haug_hbm.at[a2_v.at[p]], rows_v.at[p], sems[p][0]).wait()
        pltpu.make_async_copy(
            spa_hbm.at[a0_v.at[p]], spa0_v.at[p], sems[p][1]).wait()
        pltpu.make_async_copy(
            sprt_hbm.at[r1_v.at[p]], spr_v.at[p], sems[p][2]).wait()

    def compute_scatter(p):
        @plsc.parallel_loop(0, CH, 1, unroll=8)
        def _edge_loop(c):
            ehv = (spa0_v[p, c, pl.ds(0, 16)]
                   + rows_v[p, c, pl.ds(FH, 16)]
                   + spr_v[p, c, pl.ds(0, 16)])
            ev = jnp.exp(-jnp.where(ehv >= 0, ehv, 0.2 * ehv))
            e0b = jnp.broadcast_to(ev[0], (16,))
            e1b = jnp.broadcast_to(ev[1], (16,))
            for j in range(FH // 16):
                sl = pl.ds(j * 16, 16)
                row = rows_v[p, c, sl]
                valf0_v[c, sl] = row * e0b
                valf1_v[c, sl] = row * e1b
            valf0_v[c, pl.ds(FH, 16)] = jnp.where(lane < 2, ev, 0.0)

        idx = a0_v.at[p]
        pltpu.sync_copy(valf0_v, accf0.at[idx], add=True)
        pltpu.sync_copy(valf1_v, accf1.at[idx], add=True)

    wait_idx_done = wait_idx

    def round_body(g, p):
        @pl.when(g + 1 < NCHUNK)
        def _():
            wait_idx_done(g + 1, 1 - p)
            issue_gathers(1 - p)
        wait_gathers(p)
        compute_scatter(p)

        @pl.when(g + 2 < NCHUNK)
        def _():
            issue_idx(g + 2, p)

    issue_idx(0, 0)
    wait_idx(0, 0)
    issue_gathers(0)
    issue_idx(1, 1)

    def pair(gp, _):
        g0 = gp * 2
        round_body(g0, 0)
        round_body(g0 + 1, 1)
        return 0

    lax.fori_loop(0, NCHUNK // 2, pair, 0)
    plsc.subcore_barrier()

    @pl.when(s_id < 10)
    def _():
        src = s_id * 1000
        off = c_id * N + src
        pltpu.sync_copy(accf0.at[pl.ds(src, 1000)],
                        rawf0_hbm.at[pl.ds(off, 1000)])

    @pl.when(s_id >= 6)
    def _():
        src = (s_id - 6) * 1000
        off = c_id * N + src
        pltpu.sync_copy(accf1.at[pl.ds(src, 1000)],
                        rawf1_hbm.at[pl.ds(off, 1000)])


def kernel(input, inputr, A, w, a_src_dst):
    x = input.astype(jnp.float32)
    a0 = A[0].astype(jnp.int32)
    a2 = A[2].astype(jnp.int32)
    r1 = inputr[1].astype(jnp.int32)
    w0 = w[0].astype(jnp.float32).reshape(1, F)
    af = a_src_dst.astype(jnp.float32)[:, :, :, 0]
    ams = [jnp.pad(jnp.swapaxes(af[:, k, :], 0, 1), ((0, 0), (0, SPW - 2)))
           for k in range(3)]

    b0 = 1000
    haug, spa, sprt = pl.pallas_call(
        _prep_body,
        grid=(N // b0,),
        in_specs=[
            pl.BlockSpec((b0, F), lambda i: (i, 0)),
            pl.BlockSpec((1, F), lambda i: (0, 0)),
            pl.BlockSpec((F, SPW), lambda i: (0, 0)),
            pl.BlockSpec((F, SPW), lambda i: (0, 0)),
            pl.BlockSpec((F, SPW), lambda i: (0, 0)),
        ],
        out_specs=[
            pl.BlockSpec((2, b0, RW), lambda i: (0, i, 0)),
            pl.BlockSpec((b0, SPW), lambda i: (i, 0)),
            pl.BlockSpec((b0, SPW), lambda i: (i, 0)),
        ],
        out_shape=[
            jax.ShapeDtypeStruct((2, N, RW), jnp.float32),
            jax.ShapeDtypeStruct((N, SPW), jnp.float32),
            jax.ShapeDtypeStruct((N, SPW), jnp.float32),
        ],
    )(x, w0, ams[0], ams[1], ams[2])
    haug_flat = haug.reshape(2 * N, RW)

    mesh = plsc.VectorSubcoreMesh(
        core_axis_name="c", subcore_axis_name="s",
        num_cores=NC, num_subcores=NS)
    edge_call = functools.partial(
        pl.kernel,
        out_type=[
            jax.ShapeDtypeStruct((NC * N, RW), jnp.float32),
            jax.ShapeDtypeStruct((NC * N, FH), jnp.float32),
        ],
        mesh=mesh,
        scratch_types=[
            pltpu.VMEM((2, CH), jnp.int32),
            pltpu.VMEM((2, CH), jnp.int32),
            pltpu.VMEM((2, CH), jnp.int32),
            pltpu.VMEM((2, CH, RW), jnp.float32),
            pltpu.VMEM((2, CH, SPW), jnp.float32),
            pltpu.VMEM((2, CH, SPW), jnp.float32),
            pltpu.VMEM((CH, RW), jnp.float32),
            pltpu.VMEM((CH, FH), jnp.float32),
            pltpu.VMEM_SHARED((N, RW), jnp.float32),
            pltpu.VMEM_SHARED((N, FH), jnp.float32),
            pltpu.SemaphoreType.DMA,
            pltpu.SemaphoreType.DMA,
            pltpu.SemaphoreType.DMA,
            pltpu.SemaphoreType.DMA,
            pltpu.SemaphoreType.DMA,
            pltpu.SemaphoreType.DMA,
            pltpu.SemaphoreType.DMA,
            pltpu.SemaphoreType.DMA,
        ],
        compiler_params=pltpu.CompilerParams(use_tc_tiling_on_sc=False),
    )(_edge_kernel)
    rawf0, rawf1 = edge_call(haug_flat, spa, sprt, a0, a2, r1)

    b2 = 1000
    nb = N // b2
    out = pl.pallas_call(
        _fin_body,
        grid=(nb,),
        in_specs=[
            pl.BlockSpec((b2, RW), lambda i: (i, 0)),
            pl.BlockSpec((b2, RW), lambda i: (nb + i, 0)),
            pl.BlockSpec((b2, FH), lambda i: (i, 0)),
            pl.BlockSpec((b2, FH), lambda i: (nb + i, 0)),
        ],
        out_specs=pl.BlockSpec((2, b2, F), lambda i: (0, i, 0)),
        out_shape=jax.ShapeDtypeStruct((N_HEADS, N, F), jnp.float32),
    )(rawf0, rawf0, rawf1, rawf1)
    return out


# speedup vs baseline: 2.1572x; 1.3022x over previous
import functools

import jax
import jax.numpy as jnp
from jax import lax
from jax.experimental import pallas as pl
from jax.experimental.pallas import tpu as pltpu
from jax.experimental.pallas import tpu_sc as plsc

N_HEADS = 2
N = 10000
E = 320000
F = 128
FH = 64
SPW = 16
RW = FH + SPW

NC, NS = 2, 16
EPS = E // NS
CH = 80
NCHUNK = EPS // CH


def _prep_body(x_ref, w_ref, am0_ref, am1_ref, am2_ref,
               haug_ref, spa_ref, spr_ref):
    h = x_ref[...] * w_ref[...]
    s0 = jnp.dot(h, am0_ref[...], preferred_element_type=jnp.float32)
    s1 = jnp.dot(h, am1_ref[...], preferred_element_type=jnp.float32)
    s2 = jnp.dot(h, am2_ref[...], preferred_element_type=jnp.float32)
    haug_ref[0] = jnp.concatenate([h[:, :FH], s1], axis=1)
    haug_ref[1] = jnp.concatenate([h[:, FH:], s1], axis=1)
    spa_ref[...] = s0
    spr_ref[...] = s2


def _fin_body(h0l, h0r, h1l, h1r, out_ref):
    l0 = h0l[...]
    r0 = h0r[...]
    out_ref[0] = (jnp.concatenate([l0[:, :FH], r0[:, :FH]], axis=1)
                  / l0[:, FH:FH + 1])
    out_ref[1] = (jnp.concatenate([h1l[...], h1r[...]], axis=1)
                  / l0[:, FH + 1:FH + 2])


def _edge_kernel(haug_hbm, spa_hbm, sprt_hbm, a0_hbm, a2_hbm, r1_hbm,
                 rawf0_hbm, rawf1_hbm,
                 a0_v, a2_v, r1_v, sidx_v, rows_v, spa0_v, spr_v,
                 valf0_v, valf1_v,
                 accf0, accf1,
                 sem0a, sem1a, sem2a, sem0b, sem1b, sem2b,
                 isema, isemb, ssf0a, ssf0b, ssf1):
    c_id = lax.axis_index("c")
    s_id = lax.axis_index("s")
    zero16 = jnp.zeros((16,), jnp.float32)
    lane = lax.iota(jnp.int32, 16)

    def zrow(r, _):
        for j in range(RW // 16):
            valf0_v[0, r, pl.ds(j * 16, 16)] = zero16
        for j in range(FH // 16):
            valf1_v[r, pl.ds(j * 16, 16)] = zero16
        return 0
    lax.fori_loop(0, CH, zrow, 0)

    @pl.when(s_id < 10)
    def _():
        u0 = s_id * 1000
        for k in range(1000 // 40):
            dst = pl.ds(u0 + k * 40, 40)
            pltpu.sync_copy(valf0_v.at[0, pl.ds(0, 40)], accf0.at[dst])

    @pl.when(s_id >= 6)
    def _():
        u0 = (s_id - 6) * 1000
        for k in range(1000 // 40):
            dst = pl.ds(u0 + k * 40, 40)
            pltpu.sync_copy(valf1_v.at[pl.ds(0, 40)], accf1.at[dst])

    plsc.subcore_barrier()

    cbase = c_id * N
    sems = ((sem0a, sem1a, sem2a), (sem0b, sem1b, sem2b))
    isems = (isema, isemb)
    ssf0 = (ssf0a, ssf0b)

    def idx_slices(g):
        base = s_id * EPS + g * CH
        return pl.ds(base, CH)

    def issue_idx(g, p):
        esl = idx_slices(g)
        pltpu.async_copy(a0_hbm.at[esl], a0_v.at[p], isems[p])
        pltpu.async_copy(a2_hbm.at[esl], a2_v.at[p], isems[p])
        pltpu.async_copy(r1_hbm.at[esl], r1_v.at[p], isems[p])

    def wait_idx(g, p):
        esl = idx_slices(g)
        pltpu.make_async_copy(a0_hbm.at[esl], a0_v.at[p], isems[p]).wait()
        pltpu.make_async_copy(a2_hbm.at[esl], a2_v.at[p], isems[p]).wait()
        pltpu.make_async_copy(r1_hbm.at[esl], r1_v.at[p], isems[p]).wait()

    def issue_gathers(p):
        for j in range(CH // 16):
            sl = pl.ds(j * 16, 16)
            a2_v[p, sl] = a2_v[p, sl] + cbase
        pltpu.async_copy(haug_hbm.at[a2_v.at[p]], rows_v.at[p], sems[p][0])
        pltpu.async_copy(spa_hbm.at[a0_v.at[p]], spa0_v.at[p], sems[p][1])
        pltpu.async_copy(sprt_hbm.at[r1_v.at[p]], spr_v.at[p], sems[p][2])

    def wait_gathers(p):
        pltpu.make_async_copy(
            haug_hbm.at[a2_v.at[p]], rows_v.at[p], sems[p][0]).wait()
        pltpu.make_async_copy(
            spa_hbm.at[a0_v.at[p]], spa0_v.at[p], sems[p][1]).wait()
        pltpu.make_async_copy(
            sprt_hbm.at[r1_v.at[p]], spr_v.at[p], sems[p][2]).wait()

    def wait_scatter_f0(p):
        pltpu.make_async_copy(
            valf0_v.at[p], accf0.at[sidx_v.at[p]], ssf0[p]).wait()

    def wait_scatter_f1(p):
        pltpu.make_async_copy(
            valf1_v, accf1.at[sidx_v.at[p]], ssf1).wait()

    def compute_scatter(g, p):
        @pl.when(g >= 2)
        def _():
            wait_scatter_f0(p)

        @pl.when(g >= 1)
        def _():
            wait_scatter_f1(1 - p)

        @plsc.parallel_loop(0, CH, 1, unroll=8)
        def _edge_loop(c):
            ehv = (spa0_v[p, c, pl.ds(0, 16)]
                   + rows_v[p, c, pl.ds(FH, 16)]
                   + spr_v[p, c, pl.ds(0, 16)])
            ev = jnp.exp(-jnp.where(ehv >= 0, ehv, 0.2 * ehv))
            e0b = jnp.broadcast_to(ev[0], (16,))
            e1b = jnp.broadcast_to(ev[1], (16,))
            for j in range(FH // 16):
                sl = pl.ds(j * 16, 16)
                row = rows_v[p, c, sl]
                valf0_v[p, c, sl] = row * e0b
                valf1_v[c, sl] = row * e1b
            valf0_v[p, c, pl.ds(FH, 16)] = jnp.where(lane < 2, ev, 0.0)

        for j in range(CH // 16):
            sl = pl.ds(j * 16, 16)
            sidx_v[p, sl] = a0_v[p, sl]
        idx = sidx_v.at[p]
        pltpu.async_copy(valf0_v.at[p], accf0.at[idx], ssf0[p], add=True)
        pltpu.async_copy(valf1_v, accf1.at[idx], ssf1, add=True)

    def round_body(g, p):
        @pl.when(g + 1 < NCHUNK)
        def _():
            wait_idx(g + 1, 1 - p)
            issue_gathers(1 - p)
        wait_gathers(p)
        compute_scatter(g, p)

        @pl.when(g + 2 < NCHUNK)
        def _():
            issue_idx(g + 2, p)

    issue_idx(0, 0)
    wait_idx(0, 0)
    issue_gathers(0)
    issue_idx(1, 1)

    def pair(gp, _):
        g0 = gp * 2
        round_body(g0, 0)
        round_body(g0 + 1, 1)
        return 0

    lax.fori_loop(0, NCHUNK // 2, pair, 0)
    wait_scatter_f0(0)
    wait_scatter_f0(1)
    wait_scatter_f1(1)
    plsc.subcore_barrier()

    @pl.when(s_id < 10)
    def _():
        src = s_id * 1000
        off = c_id * N + src
        pltpu.sync_copy(accf0.at[pl.ds(src, 1000)],
                        rawf0_hbm.at[pl.ds(off, 1000)])

    @pl.when(s_id >= 6)
    def _():
        src = (s_id - 6) * 1000
        off = c_id * N + src
        pltpu.sync_copy(accf1.at[pl.ds(src, 1000)],
                        rawf1_hbm.at[pl.ds(off, 1000)])


def kernel(input, inputr, A, w, a_src_dst):
    x = input.astype(jnp.float32)
    a0 = A[0].astype(jnp.int32)
    a2 = A[2].astype(jnp.int32)
    r1 = inputr[1].astype(jnp.int32)
    w0 = w[0].astype(jnp.float32).reshape(1, F)
    af = a_src_dst.astype(jnp.float32)[:, :, :, 0]
    ams = [jnp.pad(jnp.swapaxes(af[:, k, :], 0, 1), ((0, 0), (0, SPW - 2)))
           for k in range(3)]

    b0 = 1000
    haug, spa, sprt = pl.pallas_call(
        _prep_body,
        grid=(N // b0,),
        in_specs=[
            pl.BlockSpec((b0, F), lambda i: (i, 0)),
            pl.BlockSpec((1, F), lambda i: (0, 0)),
            pl.BlockSpec((F, SPW), lambda i: (0, 0)),
            pl.BlockSpec((F, SPW), lambda i: (0, 0)),
            pl.BlockSpec((F, SPW), lambda i: (0, 0)),
        ],
        out_specs=[
            pl.BlockSpec((2, b0, RW), lambda i: (0, i, 0)),
            pl.BlockSpec((b0, SPW), lambda i: (i, 0)),
            pl.BlockSpec((b0, SPW), lambda i: (i, 0)),
        ],
        out_shape=[
            jax.ShapeDtypeStruct((2, N, RW), jnp.float32),
            jax.ShapeDtypeStruct((N, SPW), jnp.float32),
            jax.ShapeDtypeStruct((N, SPW), jnp.float32),
        ],
    )(x, w0, ams[0], ams[1], ams[2])
    haug_flat = haug.reshape(2 * N, RW)

    mesh = plsc.VectorSubcoreMesh(
        core_axis_name="c", subcore_axis_name="s",
        num_cores=NC, num_subcores=NS)
    edge_call = functools.partial(
        pl.kernel,
        out_type=[
            jax.ShapeDtypeStruct((NC * N, RW), jnp.float32),
            jax.ShapeDtypeStruct((NC * N, FH), jnp.float32),
        ],
        mesh=mesh,
        scratch_types=[
            pltpu.VMEM((2, CH), jnp.int32),
            pltpu.VMEM((2, CH), jnp.int32),
            pltpu.VMEM((2, CH), jnp.int32),
            pltpu.VMEM((2, CH), jnp.int32),
            pltpu.VMEM((2, CH, RW), jnp.float32),
            pltpu.VMEM((2, CH, SPW), jnp.float32),
            pltpu.VMEM((2, CH, SPW), jnp.float32),
            pltpu.VMEM((2, CH, RW), jnp.float32),
            pltpu.VMEM((CH, FH), jnp.float32),
            pltpu.VMEM_SHARED((N, RW), jnp.float32),
            pltpu.VMEM_SHARED((N, FH), jnp.float32),
            pltpu.SemaphoreType.DMA,
            pltpu.SemaphoreType.DMA,
            pltpu.SemaphoreType.DMA,
            pltpu.SemaphoreType.DMA,
            pltpu.SemaphoreType.DMA,
            pltpu.SemaphoreType.DMA,
            pltpu.SemaphoreType.DMA,
            pltpu.SemaphoreType.DMA,
            pltpu.SemaphoreType.DMA,
            pltpu.SemaphoreType.DMA,
            pltpu.SemaphoreType.DMA,
        ],
        compiler_params=pltpu.CompilerParams(use_tc_tiling_on_sc=False),
    )(_edge_kernel)
    rawf0, rawf1 = edge_call(haug_flat, spa, sprt, a0, a2, r1)

    b2 = 1000
    nb = N // b2
    out = pl.pallas_call(
        _fin_body,
        grid=(nb,),
        in_specs=[
            pl.BlockSpec((b2, RW), lambda i: (i, 0)),
            pl.BlockSpec((b2, RW), lambda i: (nb + i, 0)),
            pl.BlockSpec((b2, FH), lambda i: (i, 0)),
            pl.BlockSpec((b2, FH), lambda i: (nb + i, 0)),
        ],
        out_specs=pl.BlockSpec((2, b2, F), lambda i: (0, i, 0)),
        out_shape=jax.ShapeDtypeStruct((N_HEADS, N, F), jnp.float32),
    )(rawf0, rawf0, rawf1, rawf1)
    return out


# speedup vs baseline: 2.2500x; 1.0430x over previous
import functools

import jax
import jax.numpy as jnp
from jax import lax
from jax.experimental import pallas as pl
from jax.experimental.pallas import tpu as pltpu
from jax.experimental.pallas import tpu_sc as plsc

N_HEADS = 2
N = 10000
E = 320000
F = 128
FH = 64
SPW = 16
RW = FH + SPW

NC, NS = 2, 16
EPS = E // NS
CH = 80
NCHUNK = EPS // CH


def _prep_body(x_ref, w_ref, am0_ref, am1_ref, am2_ref,
               haug_ref, spa_ref, spr_ref):
    h = x_ref[...] * w_ref[...]
    s0 = jnp.dot(h, am0_ref[...], preferred_element_type=jnp.float32)
    s1 = jnp.dot(h, am1_ref[...], preferred_element_type=jnp.float32)
    s2 = jnp.dot(h, am2_ref[...], preferred_element_type=jnp.float32)
    haug_ref[0] = jnp.concatenate([h[:, :FH], s1], axis=1)
    haug_ref[1] = jnp.concatenate([h[:, FH:], s1], axis=1)
    spa_ref[...] = s0
    spr_ref[...] = s2


def _fin_body(h0l, h0r, h1l, h1r, out_ref):
    l0 = h0l[...]
    r0 = h0r[...]
    out_ref[0] = (jnp.concatenate([l0[:, :FH], r0[:, :FH]], axis=1)
                  / l0[:, FH:FH + 1])
    out_ref[1] = (jnp.concatenate([h1l[...], h1r[...]], axis=1)
                  / l0[:, FH + 1:FH + 2])


def _edge_kernel(haug_hbm, spa_hbm, sprt_hbm, a0_hbm, a2_hbm, r1_hbm,
                 rawf0_hbm, rawf1_hbm,
                 a0_v, a2_v, r1_v, sidx_v, rows_v, spa0_v, spr_v,
                 valf0_v, valf1_v,
                 accf0, accf1,
                 sem0a, sem1a, sem2a, sem0b, sem1b, sem2b,
                 isema, isemb, ssf0a, ssf0b, ssf1):
    c_id = lax.axis_index("c")
    s_id = lax.axis_index("s")
    zero16 = jnp.zeros((16,), jnp.float32)
    lane = lax.iota(jnp.int32, 16)

    def zrow(r, _):
        for j in range(RW // 16):
            valf0_v[0, r, pl.ds(j * 16, 16)] = zero16
        for j in range(FH // 16):
            valf1_v[r, pl.ds(j * 16, 16)] = zero16
        return 0
    lax.fori_loop(0, CH, zrow, 0)

    @pl.when(s_id < 10)
    def _():
        u0 = s_id * 1000
        for k in range(1000 // 40):
            dst = pl.ds(u0 + k * 40, 40)
            pltpu.sync_copy(valf0_v.at[0, pl.ds(0, 40)], accf0.at[dst])

    @pl.when(s_id >= 6)
    def _():
        u0 = (s_id - 6) * 1000
        for k in range(1000 // 40):
            dst = pl.ds(u0 + k * 40, 40)
            pltpu.sync_copy(valf1_v.at[pl.ds(0, 40)], accf1.at[dst])

    plsc.subcore_barrier()

    cbase = c_id * N
    sems = ((sem0a, sem1a, sem2a), (sem0b, sem1b, sem2b))
    isems = (isema, isemb)
    ssf0 = (ssf0a, ssf0b)

    def idx_slices(g):
        base = s_id * EPS + g * CH
        return pl.ds(base, CH)

    def issue_idx(g, p):
        esl = idx_slices(g)
        pltpu.async_copy(a0_hbm.at[esl], a0_v.at[p], isems[p])
        pltpu.async_copy(a2_hbm.at[esl], a2_v.at[p], isems[p])
        pltpu.async_copy(r1_hbm.at[esl], r1_v.at[p], isems[p])

    def wait_idx(g, p):
        esl = idx_slices(g)
        pltpu.make_async_copy(a0_hbm.at[esl], a0_v.at[p], isems[p]).wait()
        pltpu.make_async_copy(a2_hbm.at[esl], a2_v.at[p], isems[p]).wait()
        pltpu.make_async_copy(r1_hbm.at[esl], r1_v.at[p], isems[p]).wait()

    def issue_gathers(p):
        for j in range(CH // 16):
            sl = pl.ds(j * 16, 16)
            a2_v[p, sl] = a2_v[p, sl] + cbase
        pltpu.async_copy(haug_hbm.at[a2_v.at[p]], rows_v.at[p], sems[p][0])
        pltpu.async_copy(spa_hbm.at[a0_v.at[p]], spa0_v.at[p], sems[p][1])
        pltpu.async_copy(sprt_hbm.at[r1_v.at[p]], spr_v.at[p], sems[p][2])

    def wait_gathers(p):
        pltpu.make_async_copy(
            haug_hbm.at[a2_v.at[p]], rows_v.at[p], sems[p][0]).wait()
        pltpu.make_async_copy(
            spa_hbm.at[a0_v.at[p]], spa0_v.at[p], sems[p][1]).wait()
        pltpu.make_async_copy(
            sprt_hbm.at[r1_v.at[p]], spr_v.at[p], sems[p][2]).wait()

    def wait_scatter_f0(p):
        pltpu.make_async_copy(
            valf0_v.at[p], accf0.at[sidx_v.at[p]], ssf0[p]).wait()

    def wait_scatter_f1(p):
        pltpu.make_async_copy(
            valf1_v, accf1.at[sidx_v.at[p]], ssf1).wait()

    def compute_scatter(g, p):
        @pl.when(g >= 2)
        def _():
            wait_scatter_f0(p)

        @pl.when(g >= 1)
        def _():
            wait_scatter_f1(1 - p)

        @plsc.parallel_loop(0, CH, 1, unroll=16)
        def _edge_loop(c):
            ehv = (spa0_v[p, c, pl.ds(0, 16)]
                   + rows_v[p, c, pl.ds(FH, 16)]
                   + spr_v[p, c, pl.ds(0, 16)])
            ev = jnp.exp(-jnp.where(ehv >= 0, ehv, 0.2 * ehv))
            e0b = jnp.broadcast_to(ev[0], (16,))
            e1b = jnp.broadcast_to(ev[1], (16,))
            for j in range(FH // 16):
                sl = pl.ds(j * 16, 16)
                row = rows_v[p, c, sl]
                valf0_v[p, c, sl] = row * e0b
                valf1_v[c, sl] = row * e1b
            valf0_v[p, c, pl.ds(FH, 16)] = jnp.where(lane < 2, ev, 0.0)

        for j in range(CH // 16):
            sl = pl.ds(j * 16, 16)
            sidx_v[p, sl] = a0_v[p, sl]
        idx = sidx_v.at[p]
        pltpu.async_copy(valf0_v.at[p], accf0.at[idx], ssf0[p], add=True)
        pltpu.async_copy(valf1_v, accf1.at[idx], ssf1, add=True)

    def round_body(g, p):
        @pl.when(g + 1 < NCHUNK)
        def _():
            wait_idx(g + 1, 1 - p)
            issue_gathers(1 - p)
        wait_gathers(p)
        compute_scatter(g, p)

        @pl.when(g + 2 < NCHUNK)
        def _():
            issue_idx(g + 2, p)

    issue_idx(0, 0)
    wait_idx(0, 0)
    issue_gathers(0)
    issue_idx(1, 1)

    def pair(gp, _):
        g0 = gp * 2
        round_body(g0, 0)
        round_body(g0 + 1, 1)
        return 0

    lax.fori_loop(0, NCHUNK // 2, pair, 0)
    wait_scatter_f0(0)
    wait_scatter_f0(1)
    wait_scatter_f1(1)
    plsc.subcore_barrier()

    @pl.when(s_id < 10)
    def _():
        src = s_id * 1000
        off = c_id * N + src
        pltpu.sync_copy(accf0.at[pl.ds(src, 1000)],
                        rawf0_hbm.at[pl.ds(off, 1000)])

    @pl.when(s_id >= 6)
    def _():
        src = (s_id - 6) * 1000
        off = c_id * N + src
        pltpu.sync_copy(accf1.at[pl.ds(src, 1000)],
                        rawf1_hbm.at[pl.ds(off, 1000)])


def kernel(input, inputr, A, w, a_src_dst):
    x = input.astype(jnp.float32)
    a0 = A[0].astype(jnp.int32)
    a2 = A[2].astype(jnp.int32)
    r1 = inputr[1].astype(jnp.int32)
    w0 = w[0].astype(jnp.float32).reshape(1, F)
    af = a_src_dst.astype(jnp.float32)[:, :, :, 0]
    ams = [jnp.pad(jnp.swapaxes(af[:, k, :], 0, 1), ((0, 0), (0, SPW - 2)))
           for k in range(3)]

    b0 = 1000
    haug, spa, sprt = pl.pallas_call(
        _prep_body,
        grid=(N // b0,),
        in_specs=[
            pl.BlockSpec((b0, F), lambda i: (i, 0)),
            pl.BlockSpec((1, F), lambda i: (0, 0)),
            pl.BlockSpec((F, SPW), lambda i: (0, 0)),
            pl.BlockSpec((F, SPW), lambda i: (0, 0)),
            pl.BlockSpec((F, SPW), lambda i: (0, 0)),
        ],
        out_specs=[
            pl.BlockSpec((2, b0, RW), lambda i: (0, i, 0)),
            pl.BlockSpec((b0, SPW), lambda i: (i, 0)),
            pl.BlockSpec((b0, SPW), lambda i: (i, 0)),
        ],
        out_shape=[
            jax.ShapeDtypeStruct((2, N, RW), jnp.float32),
            jax.ShapeDtypeStruct((N, SPW), jnp.float32),
            jax.ShapeDtypeStruct((N, SPW), jnp.float32),
        ],
    )(x, w0, ams[0], ams[1], ams[2])
    haug_flat = haug.reshape(2 * N, RW)

    mesh = plsc.VectorSubcoreMesh(
        core_axis_name="c", subcore_axis_name="s",
        num_cores=NC, num_subcores=NS)
    edge_call = functools.partial(
        pl.kernel,
        out_type=[
            jax.ShapeDtypeStruct((NC * N, RW), jnp.float32),
            jax.ShapeDtypeStruct((NC * N, FH), jnp.float32),
        ],
        mesh=mesh,
        scratch_types=[
            pltpu.VMEM((2, CH), jnp.int32),
            pltpu.VMEM((2, CH), jnp.int32),
            pltpu.VMEM((2, CH), jnp.int32),
            pltpu.VMEM((2, CH), jnp.int32),
            pltpu.VMEM((2, CH, RW), jnp.float32),
            pltpu.VMEM((2, CH, SPW), jnp.float32),
            pltpu.VMEM((2, CH, SPW), jnp.float32),
            pltpu.VMEM((2, CH, RW), jnp.float32),
            pltpu.VMEM((CH, FH), jnp.float32),
            pltpu.VMEM_SHARED((N, RW), jnp.float32),
            pltpu.VMEM_SHARED((N, FH), jnp.float32),
            pltpu.SemaphoreType.DMA,
            pltpu.SemaphoreType.DMA,
            pltpu.SemaphoreType.DMA,
            pltpu.SemaphoreType.DMA,
            pltpu.SemaphoreType.DMA,
            pltpu.SemaphoreType.DMA,
            pltpu.SemaphoreType.DMA,
            pltpu.SemaphoreType.DMA,
            pltpu.SemaphoreType.DMA,
            pltpu.SemaphoreType.DMA,
            pltpu.SemaphoreType.DMA,
        ],
        compiler_params=pltpu.CompilerParams(use_tc_tiling_on_sc=False),
    )(_edge_kernel)
    rawf0, rawf1 = edge_call(haug_flat, spa, sprt, a0, a2, r1)

    b2 = 1000
    nb = N // b2
    out = pl.pallas_call(
        _fin_body,
        grid=(nb,),
        in_specs=[
            pl.BlockSpec((b2, RW), lambda i: (i, 0)),
            pl.BlockSpec((b2, RW), lambda i: (nb + i, 0)),
            pl.BlockSpec((b2, FH), lambda i: (i, 0)),
            pl.BlockSpec((b2, FH), lambda i: (nb + i, 0)),
        ],
        out_specs=pl.BlockSpec((2, b2, F), lambda i: (0, i, 0)),
        out_shape=jax.ShapeDtypeStruct((N_HEADS, N, F), jnp.float32),
    )(rawf0, rawf0, rawf1, rawf1)
    return out


# speedup vs baseline: 2.2864x; 1.0162x over previous
import functools

import jax
import jax.numpy as jnp
from jax import lax
from jax.experimental import pallas as pl
from jax.experimental.pallas import tpu as pltpu
from jax.experimental.pallas import tpu_sc as plsc

N_HEADS = 2
N = 10000
E = 320000
F = 128
FH = 64
SPW = 16
RW = FH + SPW

NC, NS = 2, 16
EPS = E // NS
CH = 80
NCHUNK = EPS // CH


def _prep_body(x_ref, w_ref, am0_ref, am1_ref, am2_ref,
               haug_ref, spa_ref, spr_ref):
    h = x_ref[...] * w_ref[...]
    s0 = jnp.dot(h, am0_ref[...], preferred_element_type=jnp.float32)
    s1 = jnp.dot(h, am1_ref[...], preferred_element_type=jnp.float32)
    s2 = jnp.dot(h, am2_ref[...], preferred_element_type=jnp.float32)
    haug_ref[0] = jnp.concatenate([h[:, :FH], s1], axis=1)
    haug_ref[1] = jnp.concatenate([h[:, FH:], s1], axis=1)
    spa_ref[...] = s0
    spr_ref[...] = s2


def _fin_body(h0l, h0r, h1l, h1r, out_ref):
    l0 = h0l[...]
    r0 = h0r[...]
    out_ref[0] = (jnp.concatenate([l0[:, :FH], r0[:, :FH]], axis=1)
                  / l0[:, FH:FH + 1])
    out_ref[1] = (jnp.concatenate([h1l[...], h1r[...]], axis=1)
                  / l0[:, FH + 1:FH + 2])


def _edge_kernel(haug_hbm, spa_hbm, sprt_hbm, a0_hbm, a2_hbm, r1_hbm,
                 rawf0_hbm, rawf1_hbm,
                 a0_v, a2_v, r1_v, sidx_v, rows_v, spa0_v, spr_v,
                 valf0_v, valf1_v,
                 accf0, accf1,
                 sem0a, sem1a, sem2a, sem0b, sem1b, sem2b,
                 isema, isemb, ssf0a, ssf0b, ssf1):
    c_id = lax.axis_index("c")
    s_id = lax.axis_index("s")
    zero16 = jnp.zeros((16,), jnp.float32)
    lane = lax.iota(jnp.int32, 16)

    def zrow(r, _):
        for j in range(RW // 16):
            valf0_v[0, r, pl.ds(j * 16, 16)] = zero16
        for j in range(FH // 16):
            valf1_v[r, pl.ds(j * 16, 16)] = zero16
        return 0
    lax.fori_loop(0, CH, zrow, 0)

    @pl.when(s_id < 10)
    def _():
        u0 = s_id * 1000
        for k in range(1000 // 40):
            dst = pl.ds(u0 + k * 40, 40)
            pltpu.sync_copy(valf0_v.at[0, pl.ds(0, 40)], accf0.at[dst])

    @pl.when(s_id >= 6)
    def _():
        u0 = (s_id - 6) * 1000
        for k in range(1000 // 40):
            dst = pl.ds(u0 + k * 40, 40)
            pltpu.sync_copy(valf1_v.at[pl.ds(0, 40)], accf1.at[dst])

    plsc.subcore_barrier()

    cbase = c_id * N
    sems = ((sem0a, sem1a, sem2a), (sem0b, sem1b, sem2b))
    isems = (isema, isemb)
    ssf0 = (ssf0a, ssf0b)

    def idx_slices(g):
        base = s_id * EPS + g * CH
        return pl.ds(base, CH)

    def issue_idx(g, p):
        esl = idx_slices(g)
        pltpu.async_copy(a0_hbm.at[esl], a0_v.at[p], isems[p])
        pltpu.async_copy(a2_hbm.at[esl], a2_v.at[p], isems[p])
        pltpu.async_copy(r1_hbm.at[esl], r1_v.at[p], isems[p])

    def wait_idx(g, p):
        esl = idx_slices(g)
        pltpu.make_async_copy(a0_hbm.at[esl], a0_v.at[p], isems[p]).wait()
        pltpu.make_async_copy(a2_hbm.at[esl], a2_v.at[p], isems[p]).wait()
        pltpu.make_async_copy(r1_hbm.at[esl], r1_v.at[p], isems[p]).wait()

    def issue_gathers(p):
        for j in range(CH // 16):
            sl = pl.ds(j * 16, 16)
            a2_v[p, sl] = a2_v[p, sl] + cbase
        pltpu.async_copy(haug_hbm.at[a2_v.at[p]], rows_v.at[p], sems[p][0])
        pltpu.async_copy(spa_hbm.at[a0_v.at[p]], spa0_v.at[p], sems[p][1])
        pltpu.async_copy(sprt_hbm.at[r1_v.at[p]], spr_v.at[p], sems[p][2])

    def wait_gathers(p):
        pltpu.make_async_copy(
            haug_hbm.at[a2_v.at[p]], rows_v.at[p], sems[p][0]).wait()
        pltpu.make_async_copy(
            spa_hbm.at[a0_v.at[p]], spa0_v.at[p], sems[p][1]).wait()
        pltpu.make_async_copy(
            sprt_hbm.at[r1_v.at[p]], spr_v.at[p], sems[p][2]).wait()

    def wait_scatter_f0(p):
        pltpu.make_async_copy(
            valf0_v.at[p], accf0.at[sidx_v.at[p]], ssf0[p]).wait()

    def wait_scatter_f1(p):
        pltpu.make_async_copy(
            valf1_v, accf1.at[sidx_v.at[p]], ssf1).wait()

    def compute_scatter(g, p):
        @pl.when(g >= 2)
        def _():
            wait_scatter_f0(p)

        @pl.when(g >= 1)
        def _():
            wait_scatter_f1(1 - p)

        @plsc.parallel_loop(0, CH, 1, unroll=40)
        def _edge_loop(c):
            ehv = (spa0_v[p, c, pl.ds(0, 16)]
                   + rows_v[p, c, pl.ds(FH, 16)]
                   + spr_v[p, c, pl.ds(0, 16)])
            ev = jnp.exp(-jnp.where(ehv >= 0, ehv, 0.2 * ehv))
            e0b = jnp.broadcast_to(ev[0], (16,))
            e1b = jnp.broadcast_to(ev[1], (16,))
            for j in range(FH // 16):
                sl = pl.ds(j * 16, 16)
                row = rows_v[p, c, sl]
                valf0_v[p, c, sl] = row * e0b
                valf1_v[c, sl] = row * e1b
            valf0_v[p, c, pl.ds(FH, 16)] = jnp.where(lane < 2, ev, 0.0)

        for j in range(CH // 16):
            sl = pl.ds(j * 16, 16)
            sidx_v[p, sl] = a0_v[p, sl]
        idx = sidx_v.at[p]
        pltpu.async_copy(valf0_v.at[p], accf0.at[idx], ssf0[p], add=True)
        pltpu.async_copy(valf1_v, accf1.at[idx], ssf1, add=True)

    def round_body(g, p):
        @pl.when(g + 1 < NCHUNK)
        def _():
            wait_idx(g + 1, 1 - p)
            issue_gathers(1 - p)
        wait_gathers(p)
        compute_scatter(g, p)

        @pl.when(g + 2 < NCHUNK)
        def _():
            issue_idx(g + 2, p)

    issue_idx(0, 0)
    wait_idx(0, 0)
    issue_gathers(0)
    issue_idx(1, 1)

    def pair(gp, _):
        g0 = gp * 2
        round_body(g0, 0)
        round_body(g0 + 1, 1)
        return 0

    lax.fori_loop(0, NCHUNK // 2, pair, 0)
    wait_scatter_f0(0)
    wait_scatter_f0(1)
    wait_scatter_f1(1)
    plsc.subcore_barrier()

    @pl.when(s_id < 10)
    def _():
        src = s_id * 1000
        off = c_id * N + src
        pltpu.sync_copy(accf0.at[pl.ds(src, 1000)],
                        rawf0_hbm.at[pl.ds(off, 1000)])

    @pl.when(s_id >= 6)
    def _():
        src = (s_id - 6) * 1000
        off = c_id * N + src
        pltpu.sync_copy(accf1.at[pl.ds(src, 1000)],
                        rawf1_hbm.at[pl.ds(off, 1000)])


def kernel(input, inputr, A, w, a_src_dst):
    x = input.astype(jnp.float32)
    a0 = A[0].astype(jnp.int32)
    a2 = A[2].astype(jnp.int32)
    r1 = inputr[1].astype(jnp.int32)
    w0 = w[0].astype(jnp.float32).reshape(1, F)
    af = a_src_dst.astype(jnp.float32)[:, :, :, 0]
    ams = [jnp.pad(jnp.swapaxes(af[:, k, :], 0, 1), ((0, 0), (0, SPW - 2)))
           for k in range(3)]

    b0 = 1000
    haug, spa, sprt = pl.pallas_call(
        _prep_body,
        grid=(N // b0,),
        in_specs=[
            pl.BlockSpec((b0, F), lambda i: (i, 0)),
            pl.BlockSpec((1, F), lambda i: (0, 0)),
            pl.BlockSpec((F, SPW), lambda i: (0, 0)),
            pl.BlockSpec((F, SPW), lambda i: (0, 0)),
            pl.BlockSpec((F, SPW), lambda i: (0, 0)),
        ],
        out_specs=[
            pl.BlockSpec((2, b0, RW), lambda i: (0, i, 0)),
            pl.BlockSpec((b0, SPW), lambda i: (i, 0)),
            pl.BlockSpec((b0, SPW), lambda i: (i, 0)),
        ],
        out_shape=[
            jax.ShapeDtypeStruct((2, N, RW), jnp.float32),
            jax.ShapeDtypeStruct((N, SPW), jnp.float32),
            jax.ShapeDtypeStruct((N, SPW), jnp.float32),
        ],
    )(x, w0, ams[0], ams[1], ams[2])
    haug_flat = haug.reshape(2 * N, RW)

    mesh = plsc.VectorSubcoreMesh(
        core_axis_name="c", subcore_axis_name="s",
        num_cores=NC, num_subcores=NS)
    edge_call = functools.partial(
        pl.kernel,
        out_type=[
            jax.ShapeDtypeStruct((NC * N, RW), jnp.float32),
            jax.ShapeDtypeStruct((NC * N, FH), jnp.float32),
        ],
        mesh=mesh,
        scratch_types=[
            pltpu.VMEM((2, CH), jnp.int32),
            pltpu.VMEM((2, CH), jnp.int32),
            pltpu.VMEM((2, CH), jnp.int32),
            pltpu.VMEM((2, CH), jnp.int32),
            pltpu.VMEM((2, CH, RW), jnp.float32),
            pltpu.VMEM((2, CH, SPW), jnp.float32),
            pltpu.VMEM((2, CH, SPW), jnp.float32),
            pltpu.VMEM((2, CH, RW), jnp.float32),
            pltpu.VMEM((CH, FH), jnp.float32),
            pltpu.VMEM_SHARED((N, RW), jnp.float32),
            pltpu.VMEM_SHARED((N, FH), jnp.float32),
            pltpu.SemaphoreType.DMA,
            pltpu.SemaphoreType.DMA,
            pltpu.SemaphoreType.DMA,
            pltpu.SemaphoreType.DMA,
            pltpu.SemaphoreType.DMA,
            pltpu.SemaphoreType.DMA,
            pltpu.SemaphoreType.DMA,
            pltpu.SemaphoreType.DMA,
            pltpu.SemaphoreType.DMA,
            pltpu.SemaphoreType.DMA,
            pltpu.SemaphoreType.DMA,
        ],
        compiler_params=pltpu.CompilerParams(use_tc_tiling_on_sc=False),
    )(_edge_kernel)
    rawf0, rawf1 = edge_call(haug_flat, spa, sprt, a0, a2, r1)

    b2 = 1000
    nb = N // b2
    out = pl.pallas_call(
        _fin_body,
        grid=(nb,),
        in_specs=[
            pl.BlockSpec((b2, RW), lambda i: (i, 0)),
            pl.BlockSpec((b2, RW), lambda i: (nb + i, 0)),
            pl.BlockSpec((b2, FH), lambda i: (i, 0)),
            pl.BlockSpec((b2, FH), lambda i: (nb + i, 0)),
        ],
        out_specs=pl.BlockSpec((2, b2, F), lambda i: (0, i, 0)),
        out_shape=jax.ShapeDtypeStruct((N_HEADS, N, F), jnp.float32),
    )(rawf0, rawf0, rawf1, rawf1)
    return out
